# Initial kernel scaffold; baseline (speedup 1.0000x reference)
import functools

import jax
import jax.numpy as jnp
from jax import lax
from jax.experimental import pallas as pl
from jax.experimental.pallas import tpu as pltpu
from jax.experimental.pallas import tpu_sc as plsc

_B = 32
_H = 512
_W = 512
_C = 3
_ROWS_PER_IMG = 128
_PIX_PER_ROW = 128
_ROW_WORDS = _W * _C
_K = 16
_NCHUNKS = _ROWS_PER_IMG // _K
_NBINS = 256
_N_VALS = _B * _ROWS_PER_IMG * _PIX_PER_ROW


def _sc_body(inp_hbm, parts_hbm, buf, hist, sem):
    cid = lax.axis_index("c")
    sid = lax.axis_index("s")
    wid = sid * 2 + cid

    lanes = lax.iota(jnp.int32, 16)
    ones = jnp.ones((16,), jnp.float32)

    zv = jnp.zeros((16,), jnp.float32)
    for l in range(16):
        for j in range(_NBINS // 16):
            hist[l, pl.ds(j * 16, 16)] = zv

    row_base = wid * _H

    def chunk_body(k, carry):
        row_idx = row_base + 64 * k + 4 * lanes
        pltpu.async_copy(inp_hbm.at[row_idx], buf, sem).wait()

        for r in range(_K):
            row_sel = jnp.full((16,), r, jnp.int32)
            for g in range(_PIX_PER_ROW // 16):
                col0 = g * 192 + 12 * lanes
                a = plsc.load_gather(buf, [row_sel, col0])
                b = plsc.load_gather(buf, [row_sel, col0 + 1])
                c = plsc.load_gather(buf, [row_sel, col0 + 2])
                v = (a + b + c) / 3.0
                bins = (v * 255.0).astype(jnp.int32)
                bins = jnp.minimum(jnp.maximum(bins, 0), 254)
                plsc.addupdate_scatter(hist, [lanes, bins], ones)
        return carry

    lax.fori_loop(0, _NCHUNKS, chunk_body, 0)

    pltpu.sync_copy(hist, parts_hbm.at[pl.ds(wid * 16, 16), :])


def _tc_body(parts_ref, out_ref, psedo_ref, step_ref, loss_ref):
    parts = parts_ref[:, :]
    hist = jnp.sum(parts, axis=0, keepdims=True)

    cs = hist
    for k in (1, 2, 4, 8, 16, 32, 64, 128):
        cs = cs + jnp.concatenate(
            [jnp.zeros((1, k), jnp.float32), cs[:, : _NBINS - k]], axis=1
        )
    cdf = cs[:, :255] * (1.0 / float(_N_VALS))

    out = out_ref[:, :]
    ps = psedo_ref[:, :]

    d = out - cdf
    cl = jnp.mean(d * d)
    dp = ps - out
    dc = ps - cdf
    pcl = jnp.mean(dp * dp) + 0.01 * jnp.mean(dc * dc)
    sm = out[:, 1:] - out[:, :-1]
    smooth = jnp.mean(sm * sm)

    cl_w = jnp.where(step_ref[0, 0] >= 3000, jnp.float32(0.1), jnp.float32(1.0))
    loss_ref[0, 0] = cl_w * cl + 0.1 * smooth + 0.5 * pcl


@jax.jit
def _run(output, inp, psedo_curve, step_arr):
    inp_rows = jnp.reshape(inp, (_B * _H, _ROW_WORDS))

    mesh = plsc.VectorSubcoreMesh(core_axis_name="c", subcore_axis_name="s")
    sc_hist = pl.kernel(
        _sc_body,
        out_type=jax.ShapeDtypeStruct((_B * 16, _NBINS), jnp.float32),
        mesh=mesh,
        scratch_types=[
            pltpu.VMEM((_K, _ROW_WORDS), jnp.float32),
            pltpu.VMEM((16, _NBINS), jnp.float32),
            pltpu.SemaphoreType.DMA,
        ],
    )
    parts = sc_hist(inp_rows)

    loss = pl.pallas_call(
        _tc_body,
        out_shape=jax.ShapeDtypeStruct((1, 1), jnp.float32),
        in_specs=[
            pl.BlockSpec(memory_space=pltpu.VMEM),
            pl.BlockSpec(memory_space=pltpu.VMEM),
            pl.BlockSpec(memory_space=pltpu.VMEM),
            pl.BlockSpec(memory_space=pltpu.SMEM),
        ],
        out_specs=pl.BlockSpec(memory_space=pltpu.VMEM),
    )(parts, output, psedo_curve, step_arr)

    return jnp.reshape(loss, ())


def kernel(output, inp, psedo_curve, step):
    step_arr = jnp.asarray(step, jnp.int32).reshape(1, 1)
    return _run(output, inp, psedo_curve, step_arr)


# speedup vs baseline: 3.4812x; 3.4812x over previous
import functools

import jax
import jax.numpy as jnp
from jax import lax
from jax.experimental import pallas as pl
from jax.experimental.pallas import tpu as pltpu
from jax.experimental.pallas import tpu_sc as plsc

_B = 32
_H = 512
_W = 512
_C = 3
_ROWS_PER_IMG = 128
_PIX_PER_ROW = 128
_ROW_WORDS = _W * _C
_K = 16
_NCHUNKS = _ROWS_PER_IMG // _K
_NBINS = 256
_N_VALS = _B * _ROWS_PER_IMG * _PIX_PER_ROW


def _sc_body(inp_hbm, parts_hbm, buf, hist, sem):
    cid = lax.axis_index("c")
    sid = lax.axis_index("s")
    wid = sid * 2 + cid

    lanes = lax.iota(jnp.int32, 16)
    ones = jnp.ones((16,), jnp.float32)

    zv = jnp.zeros((16,), jnp.float32)

    def zero_body(j, carry):
        for l in range(16):
            hist[l, pl.ds(j * 16, 16)] = zv
        return carry

    lax.fori_loop(0, _NBINS // 16, zero_body, 0)

    row_base = wid * _H

    def chunk_body(k, carry):
        row_idx = row_base + 64 * k + 4 * lanes
        pltpu.async_copy(inp_hbm.at[row_idx], buf, sem).wait()

        def row_body(r, carry2):
            row_sel = jnp.full((16,), 1, jnp.int32) * r
            for g in range(_PIX_PER_ROW // 16):
                col0 = g * 192 + 12 * lanes
                a = plsc.load_gather(buf, [row_sel, col0])
                b = plsc.load_gather(buf, [row_sel, col0 + 1])
                c = plsc.load_gather(buf, [row_sel, col0 + 2])
                v = (a + b + c) / 3.0
                bins = (v * 255.0).astype(jnp.int32)
                bins = jnp.minimum(jnp.maximum(bins, 0), 254)
                plsc.addupdate_scatter(hist, [lanes, bins], ones)
            return carry2

        lax.fori_loop(0, _K, row_body, 0)
        return carry

    lax.fori_loop(0, _NCHUNKS, chunk_body, 0)

    pltpu.sync_copy(hist, parts_hbm.at[pl.ds(wid * 16, 16), :])


def _tc_body(parts_ref, out_ref, psedo_ref, step_ref, loss_ref):
    parts = parts_ref[:, :]
    hist = jnp.sum(parts, axis=0, keepdims=True)

    cs = hist
    for k in (1, 2, 4, 8, 16, 32, 64, 128):
        cs = cs + jnp.concatenate(
            [jnp.zeros((1, k), jnp.float32), cs[:, : _NBINS - k]], axis=1
        )
    cdf = cs[:, :255] * (1.0 / float(_N_VALS))

    out = out_ref[:, :]
    ps = psedo_ref[:, :]

    d = out - cdf
    cl = jnp.mean(d * d)
    dp = ps - out
    dc = ps - cdf
    pcl = jnp.mean(dp * dp) + 0.01 * jnp.mean(dc * dc)
    sm = out[:, 1:] - out[:, :-1]
    smooth = jnp.mean(sm * sm)

    cl_w = jnp.where(step_ref[0, 0] >= 3000, jnp.float32(0.1), jnp.float32(1.0))
    total = cl_w * cl + 0.1 * smooth + 0.5 * pcl
    loss_ref[:, :] = jnp.reshape(total, (1, 1))


@jax.jit
def _run(output, inp, psedo_curve, step_arr):
    inp_rows = jnp.reshape(inp, (_B * _H, _ROW_WORDS))

    mesh = plsc.VectorSubcoreMesh(core_axis_name="c", subcore_axis_name="s")
    sc_hist = pl.kernel(
        _sc_body,
        out_type=jax.ShapeDtypeStruct((_B * 16, _NBINS), jnp.float32),
        mesh=mesh,
        scratch_types=[
            pltpu.VMEM((_K, _ROW_WORDS), jnp.float32),
            pltpu.VMEM((16, _NBINS), jnp.float32),
            pltpu.SemaphoreType.DMA,
        ],
        compiler_params=pltpu.CompilerParams(needs_layout_passes=False),
    )
    parts = sc_hist(inp_rows)

    loss = pl.pallas_call(
        _tc_body,
        out_shape=jax.ShapeDtypeStruct((1, 1), jnp.float32),
        in_specs=[
            pl.BlockSpec(memory_space=pltpu.VMEM),
            pl.BlockSpec(memory_space=pltpu.VMEM),
            pl.BlockSpec(memory_space=pltpu.VMEM),
            pl.BlockSpec(memory_space=pltpu.SMEM),
        ],
        out_specs=pl.BlockSpec(memory_space=pltpu.VMEM),
    )(parts, output, psedo_curve, step_arr)

    return jnp.reshape(loss, ())


def kernel(output, inp, psedo_curve, step):
    step_arr = jnp.asarray(step, jnp.int32).reshape(1, 1)
    return _run(output, inp, psedo_curve, step_arr)


# speedup vs baseline: 17.4377x; 5.0091x over previous
import functools

import jax
import jax.numpy as jnp
from jax import lax
from jax.experimental import pallas as pl
from jax.experimental.pallas import tpu as pltpu
from jax.experimental.pallas import tpu_sc as plsc

_B = 32
_H = 512
_W = 512
_C = 3
_ROWS_PER_IMG = 128
_PIX_PER_ROW = 128
_ROW_WORDS = _W * _C
_K = 16
_NCHUNKS = _ROWS_PER_IMG // _K
_NBINS = 256
_N_VALS = _B * _ROWS_PER_IMG * _PIX_PER_ROW


def _sc_body(inp_hbm, parts_hbm, buf_a, buf_b, buf_c, hist, sem):
    cid = lax.axis_index("c")
    sid = lax.axis_index("s")
    wid = sid * 2 + cid

    lanes = lax.iota(jnp.int32, 16)
    ones = jnp.ones((16,), jnp.float32)

    zv = jnp.zeros((16,), jnp.float32)

    def zero_body(j, carry):
        for l in range(16):
            hist[l, pl.ds(j * 16, 16)] = zv
        return carry

    lax.fori_loop(0, _NBINS // 16, zero_body, 0)

    plane = wid * _C * _H

    def chunk_body(k, carry):
        h_idx = 64 * k + 4 * lanes
        cp_a = pltpu.async_copy(inp_hbm.at[plane + h_idx], buf_a, sem)
        cp_b = pltpu.async_copy(inp_hbm.at[plane + _H + h_idx], buf_b, sem)
        cp_c = pltpu.async_copy(inp_hbm.at[plane + 2 * _H + h_idx], buf_c, sem)
        cp_a.wait()
        cp_b.wait()
        cp_c.wait()

        def row_body(r, carry2):
            row_sel = jnp.full((16,), 1, jnp.int32) * r
            for g in range(_PIX_PER_ROW // 16):
                col = g * 64 + 4 * lanes
                a = plsc.load_gather(buf_a, [row_sel, col])
                b = plsc.load_gather(buf_b, [row_sel, col])
                c = plsc.load_gather(buf_c, [row_sel, col])
                v = (a + b + c) / 3.0
                bins = (v * 255.0).astype(jnp.int32)
                bins = jnp.minimum(jnp.maximum(bins, 0), 254)
                plsc.addupdate_scatter(hist, [lanes, bins], ones)
            return carry2

        lax.fori_loop(0, _K, row_body, 0)
        return carry

    lax.fori_loop(0, _NCHUNKS, chunk_body, 0)

    pltpu.sync_copy(hist, parts_hbm.at[pl.ds(wid * 16, 16), :])


def _tc_body(parts_ref, out_ref, psedo_ref, step_ref, loss_ref):
    parts = parts_ref[:, :]
    hist = jnp.sum(parts, axis=0, keepdims=True)

    cs = hist
    for k in (1, 2, 4, 8, 16, 32, 64, 128):
        cs = cs + jnp.concatenate(
            [jnp.zeros((1, k), jnp.float32), cs[:, : _NBINS - k]], axis=1
        )
    cdf = cs[:, :255] * (1.0 / float(_N_VALS))

    out = out_ref[:, :]
    ps = psedo_ref[:, :]

    d = out - cdf
    cl = jnp.mean(d * d)
    dp = ps - out
    dc = ps - cdf
    pcl = jnp.mean(dp * dp) + 0.01 * jnp.mean(dc * dc)
    sm = out[:, 1:] - out[:, :-1]
    smooth = jnp.mean(sm * sm)

    cl_w = jnp.where(step_ref[0, 0] >= 3000, jnp.float32(0.1), jnp.float32(1.0))
    total = cl_w * cl + 0.1 * smooth + 0.5 * pcl
    loss_ref[:, :] = jnp.reshape(total, (1, 1))


@jax.jit
def _run(output, inp, psedo_curve, step_arr):
    inp_rows = jnp.reshape(jnp.transpose(inp, (0, 3, 1, 2)), (_B * _C * _H, _W))

    mesh = plsc.VectorSubcoreMesh(core_axis_name="c", subcore_axis_name="s")
    sc_hist = pl.kernel(
        _sc_body,
        out_type=jax.ShapeDtypeStruct((_B * 16, _NBINS), jnp.float32),
        mesh=mesh,
        scratch_types=[
            pltpu.VMEM((_K, _W), jnp.float32),
            pltpu.VMEM((_K, _W), jnp.float32),
            pltpu.VMEM((_K, _W), jnp.float32),
            pltpu.VMEM((16, _NBINS), jnp.float32),
            pltpu.SemaphoreType.DMA,
        ],
        compiler_params=pltpu.CompilerParams(needs_layout_passes=False),
    )
    parts = sc_hist(inp_rows)

    loss = pl.pallas_call(
        _tc_body,
        out_shape=jax.ShapeDtypeStruct((1, 1), jnp.float32),
        in_specs=[
            pl.BlockSpec(memory_space=pltpu.VMEM),
            pl.BlockSpec(memory_space=pltpu.VMEM),
            pl.BlockSpec(memory_space=pltpu.VMEM),
            pl.BlockSpec(memory_space=pltpu.SMEM),
        ],
        out_specs=pl.BlockSpec(memory_space=pltpu.VMEM),
    )(parts, output, psedo_curve, step_arr)

    return jnp.reshape(loss, ())


def kernel(output, inp, psedo_curve, step):
    step_arr = jnp.asarray(step, jnp.int32).reshape(1, 1)
    return _run(output, inp, psedo_curve, step_arr)


# speedup vs baseline: 21.5483x; 1.2357x over previous
import functools

import jax
import jax.numpy as jnp
from jax import lax
from jax.experimental import pallas as pl
from jax.experimental.pallas import tpu as pltpu
from jax.experimental.pallas import tpu_sc as plsc

_B = 32
_H = 512
_W = 512
_C = 3
_ROWS_PER_IMG = 128
_PIX_PER_ROW = 128
_ROW_WORDS = _W * _C
_K = 16
_NCHUNKS = _ROWS_PER_IMG // _K
_NBINS = 256
_N_VALS = _B * _ROWS_PER_IMG * _PIX_PER_ROW


def _sc_body(inp_hbm, parts_hbm, buf_a, buf_b, buf_c, hist, sem):
    cid = lax.axis_index("c")
    sid = lax.axis_index("s")
    wid = sid * 2 + cid

    lanes = lax.iota(jnp.int32, 16)
    ones = jnp.ones((16,), jnp.float32)

    zv = jnp.zeros((16,), jnp.float32)

    def zero_body(j, carry):
        for l in range(16):
            hist[l, pl.ds(j * 16, 16)] = zv
        return carry

    lax.fori_loop(0, _NBINS // 16, zero_body, 0)

    plane = wid * _C * _H

    def start_chunk(k, slot):
        h_idx = plane + 64 * k + 4 * lanes
        for cc, buf in enumerate((buf_a, buf_b, buf_c)):
            pltpu.async_copy(inp_hbm.at[h_idx + cc * _H], buf.at[slot], sem)

    def wait_chunk(k, slot):
        h_idx = plane + 64 * k + 4 * lanes
        for cc, buf in enumerate((buf_a, buf_b, buf_c)):
            pltpu.make_async_copy(inp_hbm.at[h_idx + cc * _H], buf.at[slot], sem).wait()

    def compute_chunk(slot):
        slot_sel = jnp.full((16,), slot, jnp.int32)

        def row_body(r, carry2):
            row_sel = jnp.full((16,), 1, jnp.int32) * r
            for g in range(_PIX_PER_ROW // 16):
                col = g * 64 + 4 * lanes
                a = plsc.load_gather(buf_a, [slot_sel, row_sel, col])
                b = plsc.load_gather(buf_b, [slot_sel, row_sel, col])
                c = plsc.load_gather(buf_c, [slot_sel, row_sel, col])
                v = (a + b + c) / 3.0
                bins = (v * 255.0).astype(jnp.int32)
                bins = jnp.minimum(jnp.maximum(bins, 0), 254)
                plsc.addupdate_scatter(hist, [lanes, bins], ones)
            return carry2

        lax.fori_loop(0, _K, row_body, 0)

    start_chunk(0, 0)
    start_chunk(1, 1)

    def pipe_body(kk, carry):
        for slot in range(2):
            k = 2 * kk + slot
            wait_chunk(k, slot)
            compute_chunk(slot)

            @pl.when(kk < _NCHUNKS // 2 - 1)
            def _():
                start_chunk(k + 2, slot)

        return carry

    lax.fori_loop(0, _NCHUNKS // 2, pipe_body, 0)

    pltpu.sync_copy(hist, parts_hbm.at[pl.ds(wid * 16, 16), :])


def _tc_body(parts_ref, out_ref, psedo_ref, step_ref, loss_ref):
    parts = parts_ref[:, :]
    hist = jnp.sum(parts, axis=0, keepdims=True)

    cs = hist
    for k in (1, 2, 4, 8, 16, 32, 64, 128):
        cs = cs + jnp.concatenate(
            [jnp.zeros((1, k), jnp.float32), cs[:, : _NBINS - k]], axis=1
        )
    cdf = cs[:, :255] * (1.0 / float(_N_VALS))

    out = out_ref[:, :]
    ps = psedo_ref[:, :]

    d = out - cdf
    cl = jnp.mean(d * d)
    dp = ps - out
    dc = ps - cdf
    pcl = jnp.mean(dp * dp) + 0.01 * jnp.mean(dc * dc)
    sm = out[:, 1:] - out[:, :-1]
    smooth = jnp.mean(sm * sm)

    cl_w = jnp.where(step_ref[0, 0] >= 3000, jnp.float32(0.1), jnp.float32(1.0))
    total = cl_w * cl + 0.1 * smooth + 0.5 * pcl
    loss_ref[:, :] = jnp.reshape(total, (1, 1))


@jax.jit
def _run(output, inp, psedo_curve, step_arr):
    inp_rows = jnp.reshape(jnp.transpose(inp, (0, 3, 1, 2)), (_B * _C * _H, _W))

    mesh = plsc.VectorSubcoreMesh(core_axis_name="c", subcore_axis_name="s")
    sc_hist = pl.kernel(
        _sc_body,
        out_type=jax.ShapeDtypeStruct((_B * 16, _NBINS), jnp.float32),
        mesh=mesh,
        scratch_types=[
            pltpu.VMEM((2, _K, _W), jnp.float32),
            pltpu.VMEM((2, _K, _W), jnp.float32),
            pltpu.VMEM((2, _K, _W), jnp.float32),
            pltpu.VMEM((16, _NBINS), jnp.float32),
            pltpu.SemaphoreType.DMA,
        ],
        compiler_params=pltpu.CompilerParams(needs_layout_passes=False),
    )
    parts = sc_hist(inp_rows)

    loss = pl.pallas_call(
        _tc_body,
        out_shape=jax.ShapeDtypeStruct((1, 1), jnp.float32),
        in_specs=[
            pl.BlockSpec(memory_space=pltpu.VMEM),
            pl.BlockSpec(memory_space=pltpu.VMEM),
            pl.BlockSpec(memory_space=pltpu.VMEM),
            pl.BlockSpec(memory_space=pltpu.SMEM),
        ],
        out_specs=pl.BlockSpec(memory_space=pltpu.VMEM),
    )(parts, output, psedo_curve, step_arr)

    return jnp.reshape(loss, ())


def kernel(output, inp, psedo_curve, step):
    step_arr = jnp.asarray(step, jnp.int32).reshape(1, 1)
    return _run(output, inp, psedo_curve, step_arr)


# speedup vs baseline: 22.1888x; 1.0297x over previous
import functools

import jax
import jax.numpy as jnp
from jax import lax
from jax.experimental import pallas as pl
from jax.experimental.pallas import tpu as pltpu
from jax.experimental.pallas import tpu_sc as plsc

_B = 32
_H = 512
_W = 512
_C = 3
_ROWS_PER_IMG = 128
_PIX_PER_ROW = 128
_ROW_WORDS = _W * _C
_K = 16
_NCHUNKS = _ROWS_PER_IMG // _K
_NBINS = 256
_N_VALS = _B * _ROWS_PER_IMG * _PIX_PER_ROW


def _sc_body(inp_hbm, parts_hbm, buf_a, buf_b, buf_c, hist, sem):
    cid = lax.axis_index("c")
    sid = lax.axis_index("s")
    wid = sid * 2 + cid

    lanes = lax.iota(jnp.int32, 16)
    ones = jnp.ones((16,), jnp.float32)

    zv = jnp.zeros((16,), jnp.float32)

    def zero_body(j, carry):
        for l in range(16):
            hist[l, pl.ds(j * 16, 16)] = zv
        return carry

    lax.fori_loop(0, _NBINS // 16, zero_body, 0)

    plane = wid * _C * _H

    def start_chunk(k, slot):
        h_idx = plane + 64 * k + 4 * lanes
        for cc, buf in enumerate((buf_a, buf_b, buf_c)):
            pltpu.async_copy(inp_hbm.at[h_idx + cc * _H], buf.at[slot], sem)

    def wait_chunk(k, slot):
        h_idx = plane + 64 * k + 4 * lanes
        for cc, buf in enumerate((buf_a, buf_b, buf_c)):
            pltpu.make_async_copy(inp_hbm.at[h_idx + cc * _H], buf.at[slot], sem).wait()

    def compute_chunk(slot):
        slot_sel = jnp.full((16,), slot, jnp.int32)

        def row_body(r, carry2):
            row_sel = jnp.full((16,), 1, jnp.int32) * r
            for g in range(_PIX_PER_ROW // 16):
                col = g * 64 + 4 * lanes
                a = plsc.load_gather(buf_a, [slot_sel, row_sel, col])
                b = plsc.load_gather(buf_b, [slot_sel, row_sel, col])
                c = plsc.load_gather(buf_c, [slot_sel, row_sel, col])
                bins = ((a + b + c) * 85.0).astype(jnp.int32)
                plsc.addupdate_scatter(hist, [lanes, bins], ones)
            return carry2

        lax.fori_loop(0, _K, row_body, 0)

    start_chunk(0, 0)
    start_chunk(1, 1)
    for k in range(_NCHUNKS):
        wait_chunk(k, k % 3)
        if k + 2 < _NCHUNKS:
            start_chunk(k + 2, (k + 2) % 3)
        compute_chunk(k % 3)

    pltpu.sync_copy(hist, parts_hbm.at[pl.ds(wid * 16, 16), :])


def _tc_body(parts_ref, out_ref, psedo_ref, step_ref, loss_ref):
    parts = parts_ref[:, :]
    hist = jnp.sum(parts, axis=0, keepdims=True)

    cs = hist
    for k in (1, 2, 4, 8, 16, 32, 64, 128):
        cs = cs + jnp.concatenate(
            [jnp.zeros((1, k), jnp.float32), cs[:, : _NBINS - k]], axis=1
        )
    cdf = cs[:, :255] * (1.0 / float(_N_VALS))

    out = out_ref[:, :]
    ps = psedo_ref[:, :]

    d = out - cdf
    cl = jnp.mean(d * d)
    dp = ps - out
    dc = ps - cdf
    pcl = jnp.mean(dp * dp) + 0.01 * jnp.mean(dc * dc)
    sm = out[:, 1:] - out[:, :-1]
    smooth = jnp.mean(sm * sm)

    cl_w = jnp.where(step_ref[0, 0] >= 3000, jnp.float32(0.1), jnp.float32(1.0))
    total = cl_w * cl + 0.1 * smooth + 0.5 * pcl
    loss_ref[:, :] = jnp.reshape(total, (1, 1))


@jax.jit
def _run(output, inp, psedo_curve, step_arr):
    inp_rows = jnp.reshape(jnp.transpose(inp, (0, 3, 1, 2)), (_B * _C * _H, _W))

    mesh = plsc.VectorSubcoreMesh(core_axis_name="c", subcore_axis_name="s")
    sc_hist = pl.kernel(
        _sc_body,
        out_type=jax.ShapeDtypeStruct((_B * 16, _NBINS), jnp.float32),
        mesh=mesh,
        scratch_types=[
            pltpu.VMEM((3, _K, _W), jnp.float32),
            pltpu.VMEM((3, _K, _W), jnp.float32),
            pltpu.VMEM((3, _K, _W), jnp.float32),
            pltpu.VMEM((16, _NBINS), jnp.float32),
            pltpu.SemaphoreType.DMA,
        ],
        compiler_params=pltpu.CompilerParams(needs_layout_passes=False),
    )
    parts = sc_hist(inp_rows)

    loss = pl.pallas_call(
        _tc_body,
        out_shape=jax.ShapeDtypeStruct((1, 1), jnp.float32),
        in_specs=[
            pl.BlockSpec(memory_space=pltpu.VMEM),
            pl.BlockSpec(memory_space=pltpu.VMEM),
            pl.BlockSpec(memory_space=pltpu.VMEM),
            pl.BlockSpec(memory_space=pltpu.SMEM),
        ],
        out_specs=pl.BlockSpec(memory_space=pltpu.VMEM),
    )(parts, output, psedo_curve, step_arr)

    return jnp.reshape(loss, ())


def kernel(output, inp, psedo_curve, step):
    step_arr = jnp.asarray(step, jnp.int32).reshape(1, 1)
    return _run(output, inp, psedo_curve, step_arr)


# speedup vs baseline: 25.3678x; 1.1433x over previous
import functools

import jax
import jax.numpy as jnp
from jax import lax
from jax.experimental import pallas as pl
from jax.experimental.pallas import tpu as pltpu
from jax.experimental.pallas import tpu_sc as plsc

_B = 32
_H = 512
_W = 512
_C = 3
_ROWS_PER_IMG = 128
_PIX_PER_ROW = 128
_ROW_WORDS = _W * _C
_K = 16
_NCHUNKS = _ROWS_PER_IMG // _K
_NBINS = 256
_N_VALS = _B * _ROWS_PER_IMG * _PIX_PER_ROW


def _sc_body(inp_hbm, parts_hbm, buf_a, buf_b, buf_c, hist, sem):
    cid = lax.axis_index("c")
    sid = lax.axis_index("s")
    wid = sid * 2 + cid

    lanes = lax.iota(jnp.int32, 16)
    ones = jnp.ones((16,), jnp.float32)

    zv = jnp.zeros((16,), jnp.float32)

    def zero_body(j, carry):
        for l in range(16):
            hist[l, pl.ds(j * 16, 16)] = zv
        return carry

    lax.fori_loop(0, _NBINS // 16, zero_body, 0)

    plane = wid * _C * _H

    def start_chunk(k, slot):
        h_idx = plane + 64 * k + 4 * lanes
        for cc, buf in enumerate((buf_a, buf_b, buf_c)):
            pltpu.async_copy(inp_hbm.at[h_idx + cc * _H], buf.at[slot], sem)

    def wait_chunk(k, slot):
        h_idx = plane + 64 * k + 4 * lanes
        for cc, buf in enumerate((buf_a, buf_b, buf_c)):
            pltpu.make_async_copy(inp_hbm.at[h_idx + cc * _H], buf.at[slot], sem).wait()

    def compute_chunk(slot):
        slot_sel = jnp.full((16,), slot, jnp.int32)

        @plsc.parallel_loop(0, _K, 1, unroll=2)
        def _row(r):
            row_sel = jnp.full((16,), 1, jnp.int32) * r
            for g in range(_PIX_PER_ROW // 16):
                col = g * 64 + 4 * lanes
                a = plsc.load_gather(buf_a, [slot_sel, row_sel, col])
                b = plsc.load_gather(buf_b, [slot_sel, row_sel, col])
                c = plsc.load_gather(buf_c, [slot_sel, row_sel, col])
                bins = ((a + b + c) * 85.0).astype(jnp.int32)
                plsc.addupdate_scatter(hist, [lanes, bins], ones)

    start_chunk(0, 0)
    start_chunk(1, 1)
    for k in range(_NCHUNKS):
        wait_chunk(k, k % 3)
        if k + 2 < _NCHUNKS:
            start_chunk(k + 2, (k + 2) % 3)
        compute_chunk(k % 3)

    pltpu.sync_copy(hist, parts_hbm.at[pl.ds(wid * 16, 16), :])


def _tc_body(parts_ref, out_ref, psedo_ref, step_ref, loss_ref):
    parts = parts_ref[:, :]
    hist = jnp.sum(parts, axis=0, keepdims=True)

    cs = hist
    for k in (1, 2, 4, 8, 16, 32, 64, 128):
        cs = cs + jnp.concatenate(
            [jnp.zeros((1, k), jnp.float32), cs[:, : _NBINS - k]], axis=1
        )
    cdf = cs[:, :255] * (1.0 / float(_N_VALS))

    out = out_ref[:, :]
    ps = psedo_ref[:, :]

    d = out - cdf
    cl = jnp.mean(d * d)
    dp = ps - out
    dc = ps - cdf
    pcl = jnp.mean(dp * dp) + 0.01 * jnp.mean(dc * dc)
    sm = out[:, 1:] - out[:, :-1]
    smooth = jnp.mean(sm * sm)

    cl_w = jnp.where(step_ref[0, 0] >= 3000, jnp.float32(0.1), jnp.float32(1.0))
    total = cl_w * cl + 0.1 * smooth + 0.5 * pcl
    loss_ref[:, :] = jnp.reshape(total, (1, 1))


@jax.jit
def _run(output, inp, psedo_curve, step_arr):
    inp_rows = jnp.reshape(jnp.transpose(inp, (0, 3, 1, 2)), (_B * _C * _H, _W))

    mesh = plsc.VectorSubcoreMesh(core_axis_name="c", subcore_axis_name="s")
    sc_hist = pl.kernel(
        _sc_body,
        out_type=jax.ShapeDtypeStruct((_B * 16, _NBINS), jnp.float32),
        mesh=mesh,
        scratch_types=[
            pltpu.VMEM((3, _K, _W), jnp.float32),
            pltpu.VMEM((3, _K, _W), jnp.float32),
            pltpu.VMEM((3, _K, _W), jnp.float32),
            pltpu.VMEM((16, _NBINS), jnp.float32),
            pltpu.SemaphoreType.DMA,
        ],
        compiler_params=pltpu.CompilerParams(needs_layout_passes=False),
    )
    parts = sc_hist(inp_rows)

    loss = pl.pallas_call(
        _tc_body,
        out_shape=jax.ShapeDtypeStruct((1, 1), jnp.float32),
        in_specs=[
            pl.BlockSpec(memory_space=pltpu.VMEM),
            pl.BlockSpec(memory_space=pltpu.VMEM),
            pl.BlockSpec(memory_space=pltpu.VMEM),
            pl.BlockSpec(memory_space=pltpu.SMEM),
        ],
        out_specs=pl.BlockSpec(memory_space=pltpu.VMEM),
    )(parts, output, psedo_curve, step_arr)

    return jnp.reshape(loss, ())


def kernel(output, inp, psedo_curve, step):
    step_arr = jnp.asarray(step, jnp.int32).reshape(1, 1)
    return _run(output, inp, psedo_curve, step_arr)
